# baseline (speedup 1.0000x reference)
import functools

import jax
import jax.numpy as jnp
from jax import lax
from jax.experimental import pallas as pl
from jax.experimental.pallas import tpu as pltpu
from jax.experimental.pallas import tpu_sc as plsc

_B, _N, _M, _D = 2, 4096, 8192, 256
_MB = 2048
_NTILES = 32
_QT = (_B * _M) // _NTILES
_CH = 64
_NCHUNK = _QT // _CH
_L = 16


def _top3_body(qT_ref, pT_ref, idx_ref, w_ref):
    b = pl.program_id(0)
    q = qT_ref[0]
    p = pT_ref[0]
    s2 = jnp.sum(q * q, axis=0, keepdims=True)
    s2c = jnp.transpose(s2, (1, 0))
    d2 = jnp.sum(p * p, axis=0, keepdims=True)
    qm2 = (-2.0 * q).astype(jnp.bfloat16)
    qp2 = lax.dot_general(qm2, p.astype(jnp.bfloat16),
                          (((0,), (0,)), ((), ())),
                          preferred_element_type=jnp.float32)
    dist = jnp.maximum((s2c + d2) + qp2, 0.0)

    iotaf = lax.broadcasted_iota(jnp.int32, (_MB, _N), 1).astype(jnp.float32)
    cur = dist
    mins, args = [], []
    for k in range(3):
        mn = jnp.min(cur, axis=1, keepdims=True)
        amf = jnp.min(jnp.where(cur == mn, iotaf, jnp.float32(_N)),
                      axis=1, keepdims=True)
        mins.append(mn)
        args.append(amf.astype(jnp.int32))
        if k < 2:
            cur = jnp.where(iotaf == amf, jnp.float32(jnp.inf), cur)

    recip = [1.0 / (mn + 1e-8) for mn in mins]
    norm = recip[0] + recip[1] + recip[2]
    for k in range(3):
        wk = recip[k] / norm
        w_ref[0, k] = jnp.broadcast_to(wk, (_MB, _L))
        gk = args[k] + b * _N
        idx_ref[0, k, :] = jnp.transpose(gk, (1, 0))[0]


def _top3_call(qT, pT):
    grid = (_B, _M // _MB)
    return pl.pallas_call(
        _top3_body,
        grid=grid,
        in_specs=[
            pl.BlockSpec((1, 3, _MB), lambda b, j: (b, 0, j)),
            pl.BlockSpec((1, 3, _N), lambda b, j: (b, 0, 0)),
        ],
        out_specs=[
            pl.BlockSpec((1, 3, _MB), lambda b, j: (b, 0, j)),
            pl.BlockSpec((1, 3, _MB, _L), lambda b, j: (b, 0, j, 0)),
        ],
        out_shape=[
            jax.ShapeDtypeStruct((_B, 3, _M), jnp.int32),
            jax.ShapeDtypeStruct((_B, 3, _M, _L), jnp.float32),
        ],
        compiler_params=pltpu.CompilerParams(
            dimension_semantics=("parallel", "parallel")),
    )(qT, pT)


def _sc_gather_combine(feats2d, idx_flat, w_flat):
    mesh = plsc.VectorSubcoreMesh(core_axis_name="c", subcore_axis_name="s",
                                  num_cores=2, num_subcores=16)

    @functools.partial(
        pl.kernel,
        out_type=jax.ShapeDtypeStruct((_B, _D, _M), jnp.float32),
        mesh=mesh,
        scratch_types=[
            pltpu.VMEM((_QT,), jnp.int32),
            pltpu.VMEM((_QT,), jnp.int32),
            pltpu.VMEM((_QT,), jnp.int32),
            pltpu.VMEM((_CH, _D), jnp.float32),
            pltpu.VMEM((_CH, _D), jnp.float32),
            pltpu.VMEM((_CH, _D), jnp.float32),
            pltpu.VMEM((_D, _CH), jnp.float32),
            pltpu.VMEM((_QT, _L), jnp.float32),
            pltpu.VMEM((_QT, _L), jnp.float32),
            pltpu.VMEM((_QT, _L), jnp.float32),
            pltpu.SemaphoreType.DMA,
        ],
        compiler_params=pltpu.CompilerParams(use_tc_tiling_on_sc=False, needs_layout_passes=False),
    )
    def sc_kernel(feats_hbm, idx_hbm, w_hbm, out_hbm,
                  i0_v, i1_v, i2_v,
                  r0_v, r1_v, r2_v, acc_v,
                  w0_v, w1_v, w2_v, sem):
        wid = lax.axis_index("s") * 2 + lax.axis_index("c")
        qbase = wid * _QT
        b = qbase // _M
        qib = qbase - b * _M

        ivs = (i0_v, i1_v, i2_v)
        wvs = (w0_v, w1_v, w2_v)
        woffs = [(b * 3 + k) * _M + qib for k in range(3)]

        for k in range(3):
            pltpu.sync_copy(idx_hbm.at[pl.ds(woffs[k], _QT)], ivs[k])
            pltpu.sync_copy(w_hbm.at[pl.ds(woffs[k], _QT)], wvs[k])

        for c in range(_NCHUNK):
            g0 = pltpu.async_copy(
                feats_hbm.at[i0_v.at[pl.ds(c * _CH, _CH)]], r0_v, sem)
            g1 = pltpu.async_copy(
                feats_hbm.at[i1_v.at[pl.ds(c * _CH, _CH)]], r1_v, sem)
            g2 = pltpu.async_copy(
                feats_hbm.at[i2_v.at[pl.ds(c * _CH, _CH)]], r2_v, sem)
            g0.wait()
            g1.wait()
            g2.wait()

            def qbody(qi, _, c=c):
                qg = c * _CH + qi
                w0 = w0_v[qg]
                w1 = w1_v[qg]
                w2 = w2_v[qg]
                lane = lax.broadcasted_iota(jnp.int32, (_L,), 0)
                col = jnp.zeros((_L,), jnp.int32) + qi
                for dbo in range(0, _D, _L):
                    v = (w0 * r0_v[qi, pl.ds(dbo, _L)]
                         + w1 * r1_v[qi, pl.ds(dbo, _L)]
                         + w2 * r2_v[qi, pl.ds(dbo, _L)])
                    plsc.store_scatter(acc_v, [lane + dbo, col], v)
                return 0

            lax.fori_loop(0, _CH, qbody, 0)
            pltpu.sync_copy(
                acc_v, out_hbm.at[b, :, pl.ds(qib + c * _CH, _CH)])

    return sc_kernel(feats2d, idx_flat, w_flat)


def kernel(points, features, query_points):
    pT = jnp.transpose(points, (0, 2, 1))
    qT = jnp.transpose(query_points, (0, 2, 1))
    feats2d = jnp.transpose(features, (0, 2, 1)).reshape(_B * _N, _D)
    idxg, w = _top3_call(qT, pT)
    return _sc_gather_combine(feats2d, idxg.reshape(-1),
                              w.reshape(_B * 3 * _M, _L))


# speedup vs baseline: 1.2846x; 1.2846x over previous
import functools

import jax
import jax.numpy as jnp
from jax import lax
from jax.experimental import pallas as pl
from jax.experimental.pallas import tpu as pltpu
from jax.experimental.pallas import tpu_sc as plsc

_B, _N, _M, _D = 2, 4096, 8192, 256
_MB = 2048
_NTILES = 32
_QT = (_B * _M) // _NTILES
_CH = 64
_NCHUNK = _QT // _CH
_L = 16


def _top3_body(qT_ref, pT_ref, idx_ref, w_ref):
    b = pl.program_id(0)
    q = qT_ref[0]
    p = pT_ref[0]
    s2 = jnp.sum(q * q, axis=0, keepdims=True)
    s2c = jnp.transpose(s2, (1, 0))
    d2 = jnp.sum(p * p, axis=0, keepdims=True)
    qm2 = (-2.0 * q).astype(jnp.bfloat16)
    qp2 = lax.dot_general(qm2, p.astype(jnp.bfloat16),
                          (((0,), (0,)), ((), ())),
                          preferred_element_type=jnp.float32)
    dist = jnp.maximum((s2c + d2) + qp2, 0.0)

    iotaf = lax.broadcasted_iota(jnp.int32, (_MB, _N), 1).astype(jnp.float32)
    cur = dist
    mins, args = [], []
    for k in range(3):
        mn = jnp.min(cur, axis=1, keepdims=True)
        amf = jnp.min(jnp.where(cur == mn, iotaf, jnp.float32(_N)),
                      axis=1, keepdims=True)
        mins.append(mn)
        args.append(amf.astype(jnp.int32))
        if k < 2:
            cur = jnp.where(iotaf == amf, jnp.float32(jnp.inf), cur)

    recip = [1.0 / (mn + 1e-8) for mn in mins]
    norm = recip[0] + recip[1] + recip[2]
    for k in range(3):
        wk = recip[k] / norm
        w_ref[0, k] = jnp.broadcast_to(wk, (_MB, _L))
        gk = args[k] + b * _N
        idx_ref[0, k, :] = jnp.transpose(gk, (1, 0))[0]


def _top3_call(qT, pT):
    grid = (_B, _M // _MB)
    return pl.pallas_call(
        _top3_body,
        grid=grid,
        in_specs=[
            pl.BlockSpec((1, 3, _MB), lambda b, j: (b, 0, j)),
            pl.BlockSpec((1, 3, _N), lambda b, j: (b, 0, 0)),
        ],
        out_specs=[
            pl.BlockSpec((1, 3, _MB), lambda b, j: (b, 0, j)),
            pl.BlockSpec((1, 3, _MB, _L), lambda b, j: (b, 0, j, 0)),
        ],
        out_shape=[
            jax.ShapeDtypeStruct((_B, 3, _M), jnp.int32),
            jax.ShapeDtypeStruct((_B, 3, _M, _L), jnp.float32),
        ],
        compiler_params=pltpu.CompilerParams(
            dimension_semantics=("parallel", "parallel")),
    )(qT, pT)


def _sc_gather_combine(feats2d, idx_flat, w_flat):
    mesh = plsc.VectorSubcoreMesh(core_axis_name="c", subcore_axis_name="s",
                                  num_cores=2, num_subcores=16)

    @functools.partial(
        pl.kernel,
        out_type=jax.ShapeDtypeStruct((_B * _M, _D), jnp.float32),
        mesh=mesh,
        scratch_types=[
            pltpu.VMEM((_QT,), jnp.int32),
            pltpu.VMEM((_QT,), jnp.int32),
            pltpu.VMEM((_QT,), jnp.int32),
            pltpu.VMEM((_CH, _D), jnp.float32),
            pltpu.VMEM((_CH, _D), jnp.float32),
            pltpu.VMEM((_CH, _D), jnp.float32),
            pltpu.VMEM((_CH, _D), jnp.float32),
            pltpu.VMEM((_QT, _L), jnp.float32),
            pltpu.VMEM((_QT, _L), jnp.float32),
            pltpu.VMEM((_QT, _L), jnp.float32),
            pltpu.SemaphoreType.DMA,
        ],
        compiler_params=pltpu.CompilerParams(use_tc_tiling_on_sc=False),
    )
    def sc_kernel(feats_hbm, idx_hbm, w_hbm, out_hbm,
                  i0_v, i1_v, i2_v,
                  r0_v, r1_v, r2_v, acc_v,
                  w0_v, w1_v, w2_v, sem):
        wid = lax.axis_index("s") * 2 + lax.axis_index("c")
        qbase = wid * _QT
        b = qbase // _M
        qib = qbase - b * _M

        ivs = (i0_v, i1_v, i2_v)
        wvs = (w0_v, w1_v, w2_v)
        woffs = [(b * 3 + k) * _M + qib for k in range(3)]

        for k in range(3):
            pltpu.sync_copy(idx_hbm.at[pl.ds(woffs[k], _QT)], ivs[k])
            pltpu.sync_copy(w_hbm.at[pl.ds(woffs[k], _QT)], wvs[k])

        for c in range(_NCHUNK):
            g0 = pltpu.async_copy(
                feats_hbm.at[i0_v.at[pl.ds(c * _CH, _CH)]], r0_v, sem)
            g1 = pltpu.async_copy(
                feats_hbm.at[i1_v.at[pl.ds(c * _CH, _CH)]], r1_v, sem)
            g2 = pltpu.async_copy(
                feats_hbm.at[i2_v.at[pl.ds(c * _CH, _CH)]], r2_v, sem)
            g0.wait()
            g1.wait()
            g2.wait()

            def qbody(qi, _, c=c):
                qg = c * _CH + qi
                w0 = w0_v[qg]
                w1 = w1_v[qg]
                w2 = w2_v[qg]
                for dbo in range(0, _D, _L):
                    acc_v[qi, pl.ds(dbo, _L)] = (
                        w0 * r0_v[qi, pl.ds(dbo, _L)]
                        + w1 * r1_v[qi, pl.ds(dbo, _L)]
                        + w2 * r2_v[qi, pl.ds(dbo, _L)])
                return 0

            lax.fori_loop(0, _CH, qbody, 0)
            pltpu.sync_copy(acc_v, out_hbm.at[pl.ds(qbase + c * _CH, _CH)])

    return sc_kernel(feats2d, idx_flat, w_flat)


def kernel(points, features, query_points):
    pT = jnp.transpose(points, (0, 2, 1))
    qT = jnp.transpose(query_points, (0, 2, 1))
    feats2d = jnp.transpose(features, (0, 2, 1)).reshape(_B * _N, _D)
    idxg, w = _top3_call(qT, pT)
    out2d = _sc_gather_combine(feats2d, idxg.reshape(-1),
                               w.reshape(_B * 3 * _M, _L))
    return jnp.transpose(out2d.reshape(_B, _M, _D), (0, 2, 1))
